# baseline (speedup 1.0000x reference)
import functools

import jax
import jax.numpy as jnp
from jax import lax
from jax.experimental import pallas as pl
from jax.experimental.pallas import tpu as pltpu
from jax.experimental.pallas import tpu_sc as plsc

CH = 128
EP = 128


def _build(n_cores, n_workers, n_chunks, embed, maxlen):
    per_w = n_chunks * CH
    total = n_workers * per_w
    mesh = plsc.VectorSubcoreMesh(core_axis_name="c", subcore_axis_name="s")
    nbuf = 5
    n_steps = -(-(n_chunks + 2) // nbuf)
    nvec = embed // 16

    @functools.partial(
        pl.kernel,
        out_type=jax.ShapeDtypeStruct((total, EP), jnp.float32),
        mesh=mesh,
        scratch_types=[
            pltpu.VMEM((n_chunks, CH), jnp.int32),
            pltpu.VMEM((maxlen, embed), jnp.float32),
            pltpu.VMEM((nbuf, CH, EP), jnp.float32),
        ]
        + [pltpu.SemaphoreType.DMA] * (2 * nbuf),
        compiler_params=pltpu.CompilerParams(use_tc_tiling_on_sc=False),
    )
    def kern(x_hbm, tok_hbm, pos_hbm, out_hbm, idx_v, pos_v, rows, *sems):
        gsem = sems[0:nbuf]
        osem = sems[nbuf:2 * nbuf]
        wid = lax.axis_index("s") * n_cores + lax.axis_index("c")
        base = wid * per_w

        pltpu.sync_copy(x_hbm.at[wid], idx_v)
        pltpu.sync_copy(pos_hbm, pos_v)

        def step(t0, carry):
            for k in range(nbuf):
                t = t0 * nbuf + k

                @pl.when(jnp.logical_and(t >= nbuf, t < n_chunks))
                def _():
                    pltpu.make_async_copy(
                        rows.at[k], out_hbm.at[pl.ds(0, CH)], osem[k]).wait()

                @pl.when(t < n_chunks)
                def _():
                    pltpu.async_copy(tok_hbm.at[idx_v.at[t]], rows.at[k], gsem[k])

                c2 = t - 2
                b2 = (k - 2) % nbuf

                @pl.when(jnp.logical_and(c2 >= 0, c2 < n_chunks))
                def _():
                    pltpu.make_async_copy(
                        tok_hbm.at[idx_v.at[c2]], rows.at[b2], gsem[b2]).wait()
                    pbase = lax.rem(c2 * CH, maxlen)

                    def add_rows(i, _):
                        for jj in range(4):
                            j = i * 4 + jj
                            p = pbase + j
                            p = jnp.where(p >= maxlen, p - maxlen, p)
                            for v in range(nvec):
                                sl = pl.ds(v * 16, 16)
                                plsc.addupdate(rows.at[b2, j, sl], pos_v[p, sl])
                        return _

                    lax.fori_loop(0, CH // 4, add_rows, 0)
                    pltpu.async_copy(
                        rows.at[b2], out_hbm.at[pl.ds(base + c2 * CH, CH)], osem[b2])

            return carry

        lax.fori_loop(0, n_steps, step, 0)

        for b in range(nbuf):
            pltpu.make_async_copy(
                rows.at[b], out_hbm.at[pl.ds(0, CH)], osem[b]).wait()

    return kern


def kernel(x, token_table, pos_table):
    batch, maxlen = x.shape
    vocab, embed = token_table.shape
    info = plsc.get_sparse_core_info()
    n_workers = info.num_cores * info.num_subcores
    total = batch * maxlen
    per_w = total // n_workers
    assert total % n_workers == 0 and per_w % CH == 0 and per_w % maxlen == 0
    n_chunks = per_w // CH

    tok128 = jnp.pad(token_table, ((0, 0), (0, EP - embed)))
    xr = x.reshape(n_workers, n_chunks, CH).astype(jnp.int32)
    kern = _build(info.num_cores, n_workers, n_chunks, embed, maxlen)
    out = kern(xr, tok128, pos_table)
    return out[:, :embed].reshape(batch, maxlen, embed)


# speedup vs baseline: 1.0008x; 1.0008x over previous
import functools

import jax
import jax.numpy as jnp
from jax import lax
from jax.experimental import pallas as pl
from jax.experimental.pallas import tpu as pltpu
from jax.experimental.pallas import tpu_sc as plsc

CH = 160
EP = 128


def _build(n_cores, n_workers, n_chunks, embed, maxlen):
    per_w = n_chunks * CH
    total = n_workers * per_w
    mesh = plsc.VectorSubcoreMesh(core_axis_name="c", subcore_axis_name="s")
    nbuf = 4
    n_steps = -(-(n_chunks + 2) // nbuf)
    nvec = embed // 16

    @functools.partial(
        pl.kernel,
        out_type=jax.ShapeDtypeStruct((total, EP), jnp.float32),
        mesh=mesh,
        scratch_types=[
            pltpu.VMEM((n_chunks, CH), jnp.int32),
            pltpu.VMEM((maxlen, embed), jnp.float32),
            pltpu.VMEM((nbuf, CH, EP), jnp.float32),
        ]
        + [pltpu.SemaphoreType.DMA] * (2 * nbuf),
        compiler_params=pltpu.CompilerParams(use_tc_tiling_on_sc=False),
    )
    def kern(x_hbm, tok_hbm, pos_hbm, out_hbm, idx_v, pos_v, rows, *sems):
        gsem = sems[0:nbuf]
        osem = sems[nbuf:2 * nbuf]
        wid = lax.axis_index("s") * n_cores + lax.axis_index("c")
        base = wid * per_w

        pltpu.sync_copy(x_hbm.at[wid], idx_v)
        pltpu.sync_copy(pos_hbm, pos_v)

        def step(t0, carry):
            for k in range(nbuf):
                t = t0 * nbuf + k

                @pl.when(jnp.logical_and(t >= nbuf, t < n_chunks))
                def _():
                    pltpu.make_async_copy(
                        rows.at[k], out_hbm.at[pl.ds(0, CH)], osem[k]).wait()

                @pl.when(t < n_chunks)
                def _():
                    pltpu.async_copy(tok_hbm.at[idx_v.at[t]], rows.at[k], gsem[k])

                c2 = t - 2
                b2 = (k - 2) % nbuf

                @pl.when(jnp.logical_and(c2 >= 0, c2 < n_chunks))
                def _():
                    pltpu.make_async_copy(
                        tok_hbm.at[idx_v.at[c2]], rows.at[b2], gsem[b2]).wait()
                    pbase = lax.rem(c2 * CH, maxlen)

                    def add_rows(i, _):
                        for jj in range(4):
                            j = i * 4 + jj
                            p = pbase + j
                            p = jnp.where(p >= maxlen, p - maxlen, p)
                            for v in range(nvec):
                                sl = pl.ds(v * 16, 16)
                                plsc.addupdate(rows.at[b2, j, sl], pos_v[p, sl])
                        return _

                    lax.fori_loop(0, CH // 4, add_rows, 0)
                    pltpu.async_copy(
                        rows.at[b2], out_hbm.at[pl.ds(base + c2 * CH, CH)], osem[b2])

            return carry

        lax.fori_loop(0, n_steps, step, 0)

        for b in range(nbuf):
            pltpu.make_async_copy(
                rows.at[b], out_hbm.at[pl.ds(0, CH)], osem[b]).wait()

    return kern


def kernel(x, token_table, pos_table):
    batch, maxlen = x.shape
    vocab, embed = token_table.shape
    info = plsc.get_sparse_core_info()
    n_workers = info.num_cores * info.num_subcores
    total = batch * maxlen
    per_w = total // n_workers
    assert total % n_workers == 0 and per_w % CH == 0 and per_w % maxlen == 0
    n_chunks = per_w // CH

    tok128 = jnp.pad(token_table, ((0, 0), (0, EP - embed)))
    xr = x.reshape(n_workers, n_chunks, CH).astype(jnp.int32)
    kern = _build(info.num_cores, n_workers, n_chunks, embed, maxlen)
    out = kern(xr, tok128, pos_table)
    return out[:, :embed].reshape(batch, maxlen, embed)


# speedup vs baseline: 1.0639x; 1.0631x over previous
import functools

import jax
import jax.numpy as jnp
from jax import lax
from jax.experimental import pallas as pl
from jax.experimental.pallas import tpu as pltpu
from jax.experimental.pallas import tpu_sc as plsc

CH = 160
EP = 128


def _build(n_cores, n_workers, n_chunks, embed, maxlen):
    per_w = n_chunks * CH
    total = n_workers * per_w
    mesh = plsc.VectorSubcoreMesh(core_axis_name="c", subcore_axis_name="s")
    nbuf = 4
    n_steps = -(-(n_chunks + 2) // nbuf)
    import math
    pat = math.lcm(CH, maxlen)
    nrep = pat // maxlen

    @functools.partial(
        pl.kernel,
        out_type=jax.ShapeDtypeStruct((total, EP), jnp.float32),
        mesh=mesh,
        scratch_types=[
            pltpu.VMEM((n_chunks, CH), jnp.int32),
            pltpu.VMEM_SHARED((pat, EP), jnp.float32),
            pltpu.VMEM((nbuf, CH, EP), jnp.float32),
        ]
        + [pltpu.SemaphoreType.DMA] * (3 * nbuf),
        compiler_params=pltpu.CompilerParams(use_tc_tiling_on_sc=False),
    )
    def kern(x_hbm, tok_hbm, pos_hbm, out_hbm, idx_v, shpos, rows, *sems):
        psem = sems[0:nbuf]
        gsem = sems[nbuf:2 * nbuf]
        osem = sems[2 * nbuf:3 * nbuf]
        sid = lax.axis_index("s")
        wid = sid * n_cores + lax.axis_index("c")
        base = wid * per_w

        pltpu.sync_copy(x_hbm.at[wid], idx_v)

        @pl.when(sid == 0)
        def _():
            for i in range(nrep):
                pltpu.sync_copy(pos_hbm, shpos.at[pl.ds(i * maxlen, maxlen)])

        plsc.subcore_barrier()

        def step(t0, carry):
            for k in range(nbuf):
                t = t0 * nbuf + k

                @pl.when(jnp.logical_and(t >= nbuf, t < n_chunks))
                def _():
                    pltpu.make_async_copy(
                        rows.at[k], out_hbm.at[pl.ds(0, CH)], osem[k]).wait()

                @pl.when(t < n_chunks)
                def _():
                    poff = lax.rem(t * CH, pat)
                    pltpu.async_copy(
                        shpos.at[pl.ds(poff, CH)], rows.at[k], psem[k])

                c1 = t - 1
                b1 = (k - 1) % nbuf

                @pl.when(jnp.logical_and(c1 >= 0, c1 < n_chunks))
                def _():
                    poff1 = lax.rem(c1 * CH, pat)
                    pltpu.make_async_copy(
                        shpos.at[pl.ds(poff1, CH)], rows.at[b1], psem[b1]).wait()
                    pltpu.async_copy(
                        tok_hbm.at[idx_v.at[c1]], rows.at[b1], gsem[b1], add=True)

                c2 = t - 2
                b2 = (k - 2) % nbuf

                @pl.when(jnp.logical_and(c2 >= 0, c2 < n_chunks))
                def _():
                    pltpu.make_async_copy(
                        tok_hbm.at[idx_v.at[c2]], rows.at[b2], gsem[b2]).wait()
                    pltpu.async_copy(
                        rows.at[b2], out_hbm.at[pl.ds(base + c2 * CH, CH)], osem[b2])

            return carry

        lax.fori_loop(0, n_steps, step, 0)

        for b in range(nbuf):
            pltpu.make_async_copy(
                rows.at[b], out_hbm.at[pl.ds(0, CH)], osem[b]).wait()

    return kern


def kernel(x, token_table, pos_table):
    batch, maxlen = x.shape
    vocab, embed = token_table.shape
    info = plsc.get_sparse_core_info()
    n_workers = info.num_cores * info.num_subcores
    total = batch * maxlen
    per_w = total // n_workers
    assert total % n_workers == 0 and per_w % CH == 0 and per_w % maxlen == 0
    n_chunks = per_w // CH

    tok128 = jnp.pad(token_table, ((0, 0), (0, EP - embed)))
    pos128 = jnp.pad(pos_table, ((0, 0), (0, EP - embed)))
    xr = x.reshape(n_workers, n_chunks, CH).astype(jnp.int32)
    kern = _build(info.num_cores, n_workers, n_chunks, embed, maxlen)
    out = kern(xr, tok128, pos128)
    return out[:, :embed].reshape(batch, maxlen, embed)


# speedup vs baseline: 1.2420x; 1.1674x over previous
import functools

import jax
import jax.numpy as jnp
from jax import lax
from jax.experimental import pallas as pl
from jax.experimental.pallas import tpu as pltpu
from jax.experimental.pallas import tpu_sc as plsc

CH = 160
EP = 128


def _build(n_cores, n_workers, n_chunks, embed, maxlen):
    per_w = n_chunks * CH
    total = n_workers * per_w
    mesh = plsc.VectorSubcoreMesh(core_axis_name="c", subcore_axis_name="s")
    nbuf = 4
    n_steps = -(-(n_chunks + 2) // nbuf)
    import math
    pat = math.lcm(CH, maxlen)
    nrep = pat // maxlen

    @functools.partial(
        pl.kernel,
        out_type=jax.ShapeDtypeStruct((total, EP), jnp.float32),
        mesh=mesh,
        scratch_types=[
            pltpu.VMEM((n_chunks, CH), jnp.int32),
            pltpu.VMEM_SHARED((pat, embed), jnp.float32),
            pltpu.VMEM((nbuf, CH, embed), jnp.float32),
        ]
        + [pltpu.SemaphoreType.DMA] * (3 * nbuf),
        compiler_params=pltpu.CompilerParams(use_tc_tiling_on_sc=False),
    )
    def kern(x_hbm, tok_hbm, pos_hbm, out_hbm, idx_v, shpos, rows, *sems):
        psem = sems[0:nbuf]
        gsem = sems[nbuf:2 * nbuf]
        osem = sems[2 * nbuf:3 * nbuf]
        sid = lax.axis_index("s")
        wid = sid * n_cores + lax.axis_index("c")
        base = wid * per_w

        pltpu.sync_copy(x_hbm.at[wid], idx_v)

        @pl.when(sid == 0)
        def _():
            for i in range(nrep):
                pltpu.sync_copy(pos_hbm, shpos.at[pl.ds(i * maxlen, maxlen)])

        plsc.subcore_barrier()

        def step(t0, carry):
            for k in range(nbuf):
                t = t0 * nbuf + k

                @pl.when(jnp.logical_and(t >= nbuf, t < n_chunks))
                def _():
                    pltpu.make_async_copy(
                        rows.at[k], out_hbm.at[pl.ds(0, CH), pl.ds(0, embed)],
                        osem[k]).wait()

                @pl.when(t < n_chunks)
                def _():
                    poff = lax.rem(t * CH, pat)
                    pltpu.async_copy(
                        shpos.at[pl.ds(poff, CH)], rows.at[k], psem[k])

                c1 = t - 1
                b1 = (k - 1) % nbuf

                @pl.when(jnp.logical_and(c1 >= 0, c1 < n_chunks))
                def _():
                    poff1 = lax.rem(c1 * CH, pat)
                    pltpu.make_async_copy(
                        shpos.at[pl.ds(poff1, CH)], rows.at[b1], psem[b1]).wait()
                    pltpu.async_copy(
                        tok_hbm.at[idx_v.at[c1]], rows.at[b1], gsem[b1], add=True)

                c2 = t - 2
                b2 = (k - 2) % nbuf

                @pl.when(jnp.logical_and(c2 >= 0, c2 < n_chunks))
                def _():
                    pltpu.make_async_copy(
                        tok_hbm.at[idx_v.at[c2]], rows.at[b2], gsem[b2]).wait()
                    pltpu.async_copy(
                        rows.at[b2],
                        out_hbm.at[pl.ds(base + c2 * CH, CH), pl.ds(0, embed)],
                        osem[b2])

            return carry

        lax.fori_loop(0, n_steps, step, 0)

        for b in range(nbuf):
            pltpu.make_async_copy(
                rows.at[b], out_hbm.at[pl.ds(0, CH), pl.ds(0, embed)],
                osem[b]).wait()

    return kern


def kernel(x, token_table, pos_table):
    batch, maxlen = x.shape
    vocab, embed = token_table.shape
    info = plsc.get_sparse_core_info()
    n_workers = info.num_cores * info.num_subcores
    total = batch * maxlen
    per_w = total // n_workers
    assert total % n_workers == 0 and per_w % CH == 0 and per_w % maxlen == 0
    n_chunks = per_w // CH

    tok128 = jnp.pad(token_table, ((0, 0), (0, EP - embed)))
    tok2 = tok128.reshape(2 * vocab, embed)
    xr = (x.astype(jnp.int32) * 2).reshape(n_workers, n_chunks, CH)
    kern = _build(info.num_cores, n_workers, n_chunks, embed, maxlen)
    out = kern(xr, tok2, pos_table)
    return out[:, :embed].reshape(batch, maxlen, embed)


# speedup vs baseline: 1.3306x; 1.0713x over previous
import functools

import jax
import jax.numpy as jnp
from jax import lax
from jax.experimental import pallas as pl
from jax.experimental.pallas import tpu as pltpu
from jax.experimental.pallas import tpu_sc as plsc

CH = 160
EP = 128


def _build(n_cores, n_workers, n_chunks, embed, maxlen):
    per_w = n_chunks * CH
    total = n_workers * per_w
    mesh = plsc.VectorSubcoreMesh(core_axis_name="c", subcore_axis_name="s")
    nbuf = 4
    n_steps = -(-(n_chunks + 2) // nbuf)
    import math
    pat = math.lcm(CH, maxlen)
    nrep = pat // maxlen

    @functools.partial(
        pl.kernel,
        out_type=jax.ShapeDtypeStruct((total, EP), jnp.float32),
        mesh=mesh,
        scratch_types=[
            pltpu.VMEM((n_chunks, CH), jnp.int32),
            pltpu.VMEM_SHARED((pat, embed), jnp.float32),
            pltpu.VMEM((nbuf, CH, embed), jnp.float32),
        ]
        + [pltpu.SemaphoreType.DMA] * (3 * nbuf),
        compiler_params=pltpu.CompilerParams(use_tc_tiling_on_sc=False),
    )
    def kern(x_hbm, tok_hbm, pos_hbm, out_hbm, idx_v, shpos, rows, *sems):
        psem = sems[0:nbuf]
        gsem = sems[nbuf:2 * nbuf]
        osem = sems[2 * nbuf:3 * nbuf]
        sid = lax.axis_index("s")
        wid = sid * n_cores + lax.axis_index("c")
        base = wid * per_w

        pltpu.sync_copy(x_hbm.at[wid], idx_v)

        @pl.when(sid == 0)
        def _():
            for i in range(nrep):
                pltpu.sync_copy(pos_hbm, shpos.at[pl.ds(i * maxlen, maxlen)])

        plsc.subcore_barrier()

        def step(t0, carry):
            for k in range(nbuf):
                t = t0 * nbuf + k

                @pl.when(jnp.logical_and(t >= nbuf, t < n_chunks))
                def _():
                    pltpu.make_async_copy(
                        rows.at[k], out_hbm.at[pl.ds(0, CH), pl.ds(0, embed)],
                        osem[k]).wait()

                @pl.when(t < n_chunks)
                def _():
                    poff = lax.rem(t * CH, pat)
                    pltpu.async_copy(
                        shpos.at[pl.ds(poff, CH)], rows.at[k], psem[k])

                c1 = t - 1
                b1 = (k - 1) % nbuf

                @pl.when(jnp.logical_and(c1 >= 0, c1 < n_chunks))
                def _():
                    poff1 = lax.rem(c1 * CH, pat)
                    pltpu.make_async_copy(
                        shpos.at[pl.ds(poff1, CH)], rows.at[b1], psem[b1]).wait()
                    pltpu.async_copy(
                        tok_hbm.at[idx_v.at[c1]], rows.at[b1], gsem[b1], add=True)

                c2 = t - 2
                b2 = (k - 2) % nbuf

                @pl.when(jnp.logical_and(c2 >= 0, c2 < n_chunks))
                def _():
                    pltpu.make_async_copy(
                        tok_hbm.at[idx_v.at[c2]], rows.at[b2], gsem[b2]).wait()
                    pltpu.async_copy(
                        rows.at[b2],
                        out_hbm.at[pl.ds(base + c2 * CH, CH), pl.ds(0, embed)],
                        osem[b2])

            return carry

        lax.fori_loop(0, n_steps, step, 0)

        for b in range(nbuf):
            pltpu.make_async_copy(
                rows.at[b], out_hbm.at[pl.ds(0, CH), pl.ds(0, embed)],
                osem[b]).wait()

    return kern


def _widen(table_t, vocab, embed):
    bc = 2048
    grid = -(-vocab // bc)

    def body(t_ref, o_ref):
        o_ref[:, 0:embed] = jnp.transpose(t_ref[...], (1, 0))

    return pl.pallas_call(
        body,
        grid=(grid,),
        in_specs=[pl.BlockSpec((embed, bc), lambda i: (0, i))],
        out_specs=pl.BlockSpec((bc, EP), lambda i: (i, 0)),
        out_shape=jax.ShapeDtypeStruct((vocab, EP), jnp.float32),
    )(table_t)


def kernel(x, token_table, pos_table):
    batch, maxlen = x.shape
    vocab, embed = token_table.shape
    info = plsc.get_sparse_core_info()
    n_workers = info.num_cores * info.num_subcores
    total = batch * maxlen
    per_w = total // n_workers
    assert total % n_workers == 0 and per_w % CH == 0 and per_w % maxlen == 0
    n_chunks = per_w // CH

    tok128 = _widen(token_table.T, vocab, embed)
    tok2 = tok128.reshape(2 * vocab, embed)
    xr = (x.astype(jnp.int32) * 2).reshape(n_workers, n_chunks, CH)
    kern = _build(info.num_cores, n_workers, n_chunks, embed, maxlen)
    out = kern(xr, tok2, pos_table)
    return out[:, :embed].reshape(batch, maxlen, embed)


# speedup vs baseline: 1.5674x; 1.1780x over previous
import functools

import jax
import jax.numpy as jnp
from jax import lax
from jax.experimental import pallas as pl
from jax.experimental.pallas import tpu as pltpu
from jax.experimental.pallas import tpu_sc as plsc

CH = 160
EP = 128


def _build(n_cores, n_workers, n_chunks, embed, maxlen):
    per_w = n_chunks * CH
    total = n_workers * per_w
    mesh = plsc.VectorSubcoreMesh(core_axis_name="c", subcore_axis_name="s")
    nbuf = 4
    n_steps = -(-(n_chunks + 2) // nbuf)
    import math
    pat = math.lcm(CH, maxlen)
    nrep = pat // maxlen

    @functools.partial(
        pl.kernel,
        out_type=jax.ShapeDtypeStruct((total, EP), jnp.float32),
        mesh=mesh,
        scratch_types=[
            pltpu.VMEM((n_chunks, CH), jnp.int32),
            pltpu.VMEM_SHARED((pat, embed), jnp.float32),
            pltpu.VMEM((nbuf, CH, embed), jnp.float32),
        ]
        + [pltpu.SemaphoreType.DMA] * (3 * nbuf),
        compiler_params=pltpu.CompilerParams(use_tc_tiling_on_sc=False),
    )
    def kern(x_hbm, tok_hbm, pos_hbm, out_hbm, idx_v, shpos, rows, *sems):
        psem = sems[0:nbuf]
        gsem = sems[nbuf:2 * nbuf]
        osem = sems[2 * nbuf:3 * nbuf]
        sid = lax.axis_index("s")
        wid = sid * n_cores + lax.axis_index("c")
        base = wid * per_w

        pltpu.sync_copy(x_hbm.at[wid], idx_v)

        @pl.when(sid == 0)
        def _():
            for i in range(nrep):
                pltpu.sync_copy(pos_hbm, shpos.at[pl.ds(i * maxlen, maxlen)])

        plsc.subcore_barrier()

        def step(t0, carry):
            for k in range(nbuf):
                t = t0 * nbuf + k

                @pl.when(jnp.logical_and(t >= nbuf, t < n_chunks))
                def _():
                    pltpu.make_async_copy(
                        rows.at[k], out_hbm.at[pl.ds(0, CH), pl.ds(0, embed)],
                        osem[k]).wait()

                @pl.when(t < n_chunks)
                def _():
                    poff = lax.rem(t * CH, pat)
                    pltpu.async_copy(
                        shpos.at[pl.ds(poff, CH)], rows.at[k], psem[k])

                c1 = t - 1
                b1 = (k - 1) % nbuf

                @pl.when(jnp.logical_and(c1 >= 0, c1 < n_chunks))
                def _():
                    poff1 = lax.rem(c1 * CH, pat)
                    pltpu.make_async_copy(
                        shpos.at[pl.ds(poff1, CH)], rows.at[b1], psem[b1]).wait()
                    pltpu.async_copy(
                        tok_hbm.at[idx_v.at[c1]], rows.at[b1], gsem[b1], add=True)

                c2 = t - 2
                b2 = (k - 2) % nbuf

                @pl.when(jnp.logical_and(c2 >= 0, c2 < n_chunks))
                def _():
                    pltpu.make_async_copy(
                        tok_hbm.at[idx_v.at[c2]], rows.at[b2], gsem[b2]).wait()
                    pltpu.async_copy(
                        rows.at[b2],
                        out_hbm.at[pl.ds(base + c2 * CH, CH), pl.ds(0, embed)],
                        osem[b2])

            return carry

        lax.fori_loop(0, n_steps, step, 0)

        for b in range(nbuf):
            pltpu.make_async_copy(
                rows.at[b], out_hbm.at[pl.ds(0, CH), pl.ds(0, embed)],
                osem[b]).wait()

    return kern


def _widen(table_t, vocab, embed):
    bc = 4096
    grid = -(-vocab // bc)

    def body(t_ref, o_ref):
        o_ref[:, 0:embed] = jnp.transpose(t_ref[...], (1, 0))

    return pl.pallas_call(
        body,
        grid=(grid,),
        in_specs=[pl.BlockSpec((embed, bc), lambda i: (0, i))],
        out_specs=pl.BlockSpec((bc, EP), lambda i: (i, 0)),
        out_shape=jax.ShapeDtypeStruct((vocab, EP), jnp.float32),
    )(table_t)


def kernel(x, token_table, pos_table):
    batch, maxlen = x.shape
    vocab, embed = token_table.shape
    info = plsc.get_sparse_core_info()
    n_workers = info.num_cores * info.num_subcores
    total = batch * maxlen
    per_w = total // n_workers
    assert total % n_workers == 0 and per_w % CH == 0 and per_w % maxlen == 0
    n_chunks = per_w // CH

    tok128 = _widen(token_table.T, vocab, embed)
    tok2 = tok128.reshape(2 * vocab, embed)
    xr = (x.astype(jnp.int32) * 2).reshape(n_workers, n_chunks, CH)
    kern = _build(info.num_cores, n_workers, n_chunks, embed, maxlen)
    out = kern(xr, tok2, pos_table)
    return out[:, :embed].reshape(batch, maxlen, embed)


# speedup vs baseline: 1.7520x; 1.1178x over previous
import functools

import jax
import jax.numpy as jnp
from jax import lax
from jax.experimental import pallas as pl
from jax.experimental.pallas import tpu as pltpu
from jax.experimental.pallas import tpu_sc as plsc

CH = 160
EP = 128


def _build(n_cores, n_workers, n_chunks, embed, maxlen):
    per_w = n_chunks * CH
    total = n_workers * per_w
    mesh = plsc.VectorSubcoreMesh(core_axis_name="c", subcore_axis_name="s")
    nbuf = 4
    n_steps = -(-(n_chunks + 2) // nbuf)
    import math
    pat = math.lcm(CH, maxlen)
    nrep = pat // maxlen

    @functools.partial(
        pl.kernel,
        out_type=jax.ShapeDtypeStruct((total, EP), jnp.float32),
        mesh=mesh,
        scratch_types=[
            pltpu.VMEM((n_chunks, CH), jnp.int32),
            pltpu.VMEM_SHARED((pat, embed), jnp.float32),
            pltpu.VMEM((nbuf, CH, embed), jnp.float32),
        ]
        + [pltpu.SemaphoreType.DMA] * (3 * nbuf),
        compiler_params=pltpu.CompilerParams(use_tc_tiling_on_sc=False),
    )
    def kern(x_hbm, tok_hbm, pos_hbm, out_hbm, idx_v, shpos, rows, *sems):
        psem = sems[0:nbuf]
        gsem = sems[nbuf:2 * nbuf]
        osem = sems[2 * nbuf:3 * nbuf]
        sid = lax.axis_index("s")
        wid = sid * n_cores + lax.axis_index("c")
        base = wid * per_w

        pltpu.sync_copy(x_hbm.at[wid], idx_v)

        @pl.when(sid == 0)
        def _():
            for i in range(nrep):
                pltpu.sync_copy(pos_hbm, shpos.at[pl.ds(i * maxlen, maxlen)])

        plsc.subcore_barrier()

        def step(t0, carry):
            for k in range(nbuf):
                t = t0 * nbuf + k

                @pl.when(jnp.logical_and(t >= nbuf, t < n_chunks))
                def _():
                    pltpu.make_async_copy(
                        rows.at[k], out_hbm.at[pl.ds(0, CH), pl.ds(0, embed)],
                        osem[k]).wait()

                @pl.when(t < n_chunks)
                def _():
                    poff = lax.rem(t * CH, pat)
                    pltpu.async_copy(
                        shpos.at[pl.ds(poff, CH)], rows.at[k], psem[k])

                c1 = t - 1
                b1 = (k - 1) % nbuf

                @pl.when(jnp.logical_and(c1 >= 0, c1 < n_chunks))
                def _():
                    poff1 = lax.rem(c1 * CH, pat)
                    pltpu.make_async_copy(
                        shpos.at[pl.ds(poff1, CH)], rows.at[b1], psem[b1]).wait()
                    pltpu.async_copy(
                        tok_hbm.at[idx_v.at[c1]], rows.at[b1], gsem[b1], add=True)

                c2 = t - 2
                b2 = (k - 2) % nbuf

                @pl.when(jnp.logical_and(c2 >= 0, c2 < n_chunks))
                def _():
                    pltpu.make_async_copy(
                        tok_hbm.at[idx_v.at[c2]], rows.at[b2], gsem[b2]).wait()
                    pltpu.async_copy(
                        rows.at[b2],
                        out_hbm.at[pl.ds(base + c2 * CH, CH), pl.ds(0, embed)],
                        osem[b2])

            return carry

        lax.fori_loop(0, n_steps, step, 0)

        for b in range(nbuf):
            pltpu.make_async_copy(
                rows.at[b], out_hbm.at[pl.ds(0, CH), pl.ds(0, embed)],
                osem[b]).wait()

    return kern


def _widen(table_t, vocab, embed):
    bc = 8192
    grid = -(-vocab // bc)

    def body(t_ref, o_ref):
        o_ref[:, 0:embed] = jnp.transpose(t_ref[...], (1, 0))

    return pl.pallas_call(
        body,
        grid=(grid,),
        in_specs=[pl.BlockSpec((embed, bc), lambda i: (0, i))],
        out_specs=pl.BlockSpec((bc, EP), lambda i: (i, 0)),
        out_shape=jax.ShapeDtypeStruct((vocab, EP), jnp.float32),
    )(table_t)


def kernel(x, token_table, pos_table):
    batch, maxlen = x.shape
    vocab, embed = token_table.shape
    info = plsc.get_sparse_core_info()
    n_workers = info.num_cores * info.num_subcores
    total = batch * maxlen
    per_w = total // n_workers
    assert total % n_workers == 0 and per_w % CH == 0 and per_w % maxlen == 0
    n_chunks = per_w // CH

    tok128 = _widen(token_table.T, vocab, embed)
    tok2 = tok128.reshape(2 * vocab, embed)
    xr = (x.astype(jnp.int32) * 2).reshape(n_workers, n_chunks, CH)
    kern = _build(info.num_cores, n_workers, n_chunks, embed, maxlen)
    out = kern(xr, tok2, pos_table)
    return out[:, :embed].reshape(batch, maxlen, embed)
